# baseline (speedup 1.0000x reference)
import functools

import jax
import jax.numpy as jnp
from jax import lax
from jax.experimental import pallas as pl
from jax.experimental.pallas import tpu as pltpu
from jax.experimental.pallas import tpu_sc as plsc

B = 4096
H = 200
D = 64
NC = 2
NS = 16
NW = NC * NS
BPW = B // NW
IDXW = BPW * H
C0, C1 = 104, 96
LANES = 16
NG = D // LANES
RU = 4
NBUF = 4


def _pool_body(x_hbm, table_hbm, out_hbm, idx_v, buf_v, out_v,
               sem0, sem1, sem2, sem3):
    wid = lax.axis_index("s") * NC + lax.axis_index("c")
    base_b = wid * BPW
    pltpu.sync_copy(x_hbm.at[pl.ds(base_b * H, IDXW)], idx_v)

    sems = (sem0, sem1, sem2, sem3)

    def issue(b, slot):
        off = b * H
        pltpu.async_copy(table_hbm.at[idx_v.at[pl.ds(off, C0)]],
                         buf_v.at[slot, pl.ds(0, C0)], sems[slot],
                         priority=1)
        pltpu.async_copy(table_hbm.at[idx_v.at[pl.ds(off + C0, C1)]],
                         buf_v.at[slot, pl.ds(C0, C1)], sems[slot],
                         priority=1)

    for k in range(NBUF):
        issue(k, k)

    def outer(bb, carry):
        for k in range(NBUF):
            b = bb * NBUF + k
            pltpu.make_async_copy(table_hbm.at[pl.ds(0, H)],
                                  buf_v.at[k], sems[k]).wait()
            bk = buf_v.at[k]

            def acc_step(r, accs):
                new = list(accs)
                for u in range(RU):
                    for g in range(NG):
                        new[g] = new[g] + bk[r * RU + u, pl.ds(g * LANES, LANES)]
                return tuple(new)

            z = jnp.zeros((LANES,), jnp.float32)
            accs = lax.fori_loop(0, H // RU, acc_step, (z,) * NG)
            for g in range(NG):
                out_v[b, pl.ds(g * LANES, LANES)] = accs[g]

            @pl.when(b + NBUF < BPW)
            def _():
                issue(b + NBUF, k)
        return carry

    lax.fori_loop(0, BPW // NBUF, outer, 0)
    pltpu.sync_copy(out_v, out_hbm.at[pl.ds(base_b, BPW)])


def _pool(x_flat, table):
    mesh = plsc.VectorSubcoreMesh(core_axis_name="c", subcore_axis_name="s",
                                  num_cores=NC, num_subcores=NS)
    return pl.kernel(
        _pool_body,
        out_type=jax.ShapeDtypeStruct((B, D), jnp.float32),
        mesh=mesh,
        scratch_types=[
            pltpu.VMEM((IDXW,), jnp.int32),
            pltpu.VMEM((NBUF, H, D), jnp.float32),
            pltpu.VMEM((BPW, D), jnp.float32),
            pltpu.SemaphoreType.DMA,
            pltpu.SemaphoreType.DMA,
            pltpu.SemaphoreType.DMA,
            pltpu.SemaphoreType.DMA,
        ],
        compiler_params=pltpu.CompilerParams(use_tc_tiling_on_sc=False),
    )(x_flat, table)


def _mlp_body(x_ref, w1_ref, b1_ref, w2_ref, b2_ref, o_ref):
    h = lax.dot_general(x_ref[...], w1_ref[...], (((1,), (1,)), ((), ())),
                        preferred_element_type=jnp.float32)
    h = jnp.maximum(h + b1_ref[...], 0.0)
    o = lax.dot_general(h, w2_ref[...], (((1,), (1,)), ((), ())),
                        preferred_element_type=jnp.float32)
    o_ref[...] = o + b2_ref[...]


def _mlp(pooled, W1, b1, W2, b2):
    return pl.pallas_call(
        _mlp_body,
        out_shape=jax.ShapeDtypeStruct((B, D), jnp.float32),
    )(pooled, W1, b1.reshape(1, D), W2, b2.reshape(1, D))


def kernel(x, table, W1, b1, W2, b2):
    x_flat = x.reshape(-1).astype(jnp.int32)
    pooled = _pool(x_flat, table)
    out = _mlp(pooled, W1, b1, W2, b2)
    return out[None, :, :]
